# baseline (speedup 1.0000x reference)
import functools

import jax
import jax.numpy as jnp
from jax import lax
from jax.experimental import pallas as pl
from jax.experimental.pallas import tpu as pltpu
from jax.experimental.pallas import tpu_sc as plsc

N = 10000
E = 320000
D = 128
H = 128
C = 40

NC = 2
NS = 16
NW = NC * NS
CH = 128
NCHUNKS = E // CH
CPT = NCHUNKS // NW
CREM = NCHUNKS - CPT * NW
N_ACC = 10240
ROWS_PER_TILE = N_ACC // NS
ZR = 128
BLK = 1000
GRID = N // BLK


def _seg_sum_sc(W_edge, edge_index):
    mesh = plsc.VectorSubcoreMesh(core_axis_name="c", subcore_axis_name="s")

    @functools.partial(
        pl.kernel,
        out_type=jax.ShapeDtypeStruct((NC, N_ACC, H), jnp.float32),
        mesh=mesh,
        scratch_types=[
            pltpu.VMEM((2, CH), jnp.int32),
            pltpu.VMEM((2, CH), jnp.int32),
            pltpu.VMEM((2, CH), jnp.int32),
            pltpu.VMEM((2, CH), jnp.int32),
            pltpu.VMEM((CH, H), jnp.float32),
            pltpu.VMEM((CH, H), jnp.float32),
            pltpu.VMEM_SHARED((N_ACC, H), jnp.float32),
            pltpu.SemaphoreType.DMA,
            pltpu.SemaphoreType.DMA,
            pltpu.SemaphoreType.DMA,
            pltpu.SemaphoreType.DMA,
            pltpu.SemaphoreType.DMA,
            pltpu.SemaphoreType.DMA,
            pltpu.SemaphoreType.DMA,
            pltpu.SemaphoreType.DMA,
        ],
    )
    def k(w_hbm, e_hbm, out_hbm, eidx0, eidx1, eidx2, eidx3, rows0, rows1,
          acc, seme0, seme1, seme2, seme3, semg0, semg1, sems0, sems1):
        c = lax.axis_index("c")
        s = lax.axis_index("s")
        eidx = (eidx0, eidx1, eidx2, eidx3)
        rows = (rows0, rows1)
        seme = (seme0, seme1, seme2, seme3)
        semg = (semg0, semg1)
        sems = (sems0, sems1)

        wid = c * NS + s
        cbase = wid * CPT + jnp.minimum(wid, CREM)
        nch = CPT + jnp.where(wid < CREM, 1, 0)

        def zero_row(r, _):
            for j in range(H // 16):
                rows0[r, pl.ds(j * 16, 16)] = jnp.zeros((16,), jnp.float32)
            return 0

        lax.fori_loop(0, CH, zero_row, 0)
        for b in range(ROWS_PER_TILE // ZR):
            pltpu.sync_copy(rows0, acc.at[pl.ds(s * ROWS_PER_TILE + b * ZR, ZR)])
        plsc.subcore_barrier()

        pltpu.sync_copy(e_hbm.at[:, pl.ds(cbase * CH, CH)], eidx0)
        for b in range(1, 4):
            pltpu.async_copy(e_hbm.at[:, pl.ds((cbase + b) * CH, CH)],
                             eidx[b], seme[b])
        pltpu.async_copy(w_hbm.at[eidx0.at[0]], rows0, semg0)

        def step(k_i, _):
            b4 = lax.rem(k_i, 4)
            for bb in range(4):
                rb = bb % 2

                @pl.when(b4 == bb)
                def _():
                    @pl.when(k_i >= 1)
                    def _():
                        pltpu.make_async_copy(
                            rows[1 - rb], acc.at[eidx[(bb + 3) % 4].at[1]],
                            sems[1 - rb]).wait()

                        @pl.when(k_i + 3 < nch)
                        def _():
                            pltpu.async_copy(
                                e_hbm.at[:, pl.ds((cbase + k_i + 3) * CH, CH)],
                                eidx[(bb + 3) % 4], seme[(bb + 3) % 4])

                    @pl.when(k_i + 1 < nch)
                    def _():
                        pltpu.make_async_copy(
                            e_hbm.at[:, pl.ds(0, CH)], eidx[(bb + 1) % 4],
                            seme[(bb + 1) % 4]).wait()
                        pltpu.async_copy(w_hbm.at[eidx[(bb + 1) % 4].at[0]],
                                         rows[1 - rb], semg[1 - rb])

                    pltpu.make_async_copy(w_hbm.at[pl.ds(0, CH)], rows[rb],
                                          semg[rb]).wait()
                    pltpu.async_copy(rows[rb], acc.at[eidx[bb].at[1]],
                                     sems[rb], add=True)
            return 0

        lax.fori_loop(0, nch, step, 0)

        bl = lax.rem(nch - 1, 2)
        for bb in range(2):

            @pl.when(bl == bb)
            def _():
                pltpu.make_async_copy(rows[bb], acc.at[eidx[0].at[1]],
                                      sems[bb]).wait()
        plsc.subcore_barrier()
        pltpu.sync_copy(
            acc.at[pl.ds(s * ROWS_PER_TILE, ROWS_PER_TILE)],
            out_hbm.at[c, pl.ds(s * ROWS_PER_TILE, ROWS_PER_TILE)],
        )

    return k(W_edge, edge_index)


def _tc_x(x, wnt, w2t, bn, b2):
    def body(x_ref, wnt_ref, w2t_ref, bn_ref, b2_ref, o_ref):
        xh = jnp.dot(x_ref[...], wnt_ref[...], preferred_element_type=jnp.float32) + bn_ref[...]
        o_ref[...] = xh + jnp.dot(xh, w2t_ref[...], preferred_element_type=jnp.float32) + b2_ref[...]

    full = lambda i: (0, 0)
    return pl.pallas_call(
        body,
        grid=(GRID,),
        in_specs=[
            pl.BlockSpec((BLK, D), lambda i: (i, 0)),
            pl.BlockSpec((D, H), full),
            pl.BlockSpec((H, H), full),
            pl.BlockSpec((1, H), full),
            pl.BlockSpec((1, H), full),
        ],
        out_specs=pl.BlockSpec((BLK, H), lambda i: (i, 0)),
        out_shape=jax.ShapeDtypeStruct((N, H), jnp.float32),
    )(x, wnt, w2t, bn, b2)


def _tc_main(partials, xc, w1t, wf1t, be, b1, bf1, gamma2, beta2, wf2, bf2c):
    def body(p0_ref, p1_ref, xc_ref, w1t_ref, wf1t_ref, be_ref, b1_ref,
             bf1_ref, g_ref, bb_ref, wf2_ref, bf2_ref, o_ref, h_scr, st_scr):
        i = pl.program_id(0)

        @pl.when(i < GRID)
        def _():
            a = p0_ref[0] + p1_ref[0] + be_ref[...]
            a2 = a + jnp.dot(a, w1t_ref[...], preferred_element_type=jnp.float32) + b1_ref[...]
            out = jnp.maximum(a2 + xc_ref[...], 0.0)
            h1 = jnp.dot(out, wf1t_ref[...], preferred_element_type=jnp.float32) + bf1_ref[...]
            h1 = jnp.maximum(h1, 0.0)
            h_scr[pl.ds(i * BLK, BLK), :] = h1
            upd = jnp.concatenate(
                [jnp.sum(h1, axis=0, keepdims=True),
                 jnp.sum(h1 * h1, axis=0, keepdims=True),
                 jnp.zeros((6, H), jnp.float32)], axis=0)

            @pl.when(i == 0)
            def _():
                st_scr[...] = upd

            @pl.when(i > 0)
            def _():
                st_scr[...] = st_scr[...] + upd

        @pl.when(i >= GRID)
        def _():
            j = i - GRID
            st = st_scr[...]
            mean = st[0:1, :] * (1.0 / N)
            var = st[1:2, :] * (1.0 / N) - mean * mean
            scale = lax.rsqrt(var + 1e-5) * g_ref[...]
            h1 = h_scr[pl.ds(j * BLK, BLK), :]
            hn = (h1 - mean) * scale + bb_ref[...]
            lt = lax.dot_general(hn, wf2_ref[...], (((1,), (1,)), ((), ())),
                                 preferred_element_type=jnp.float32)
            lt = lt + bf2_ref[...]
            m = jnp.max(lt, axis=1, keepdims=True)
            ex = jnp.exp(lt - m)
            lse = jnp.log(jnp.sum(ex, axis=1, keepdims=True))
            o_ref[...] = lt - m - lse

    full = lambda i: (0, 0)
    rowblk = lambda i: (jnp.minimum(i, GRID - 1), 0)
    return pl.pallas_call(
        body,
        grid=(2 * GRID,),
        in_specs=[
            pl.BlockSpec((1, BLK, H), lambda i: (0, jnp.minimum(i, GRID - 1), 0)),
            pl.BlockSpec((1, BLK, H), lambda i: (1, jnp.minimum(i, GRID - 1), 0)),
            pl.BlockSpec((BLK, H), rowblk),
            pl.BlockSpec((H, H), full),
            pl.BlockSpec((H, H), full),
            pl.BlockSpec((1, H), full),
            pl.BlockSpec((1, H), full),
            pl.BlockSpec((1, H), full),
            pl.BlockSpec((1, H), full),
            pl.BlockSpec((1, H), full),
            pl.BlockSpec((C, H), full),
            pl.BlockSpec((1, C), full),
        ],
        out_specs=pl.BlockSpec((BLK, C), lambda i: (jnp.maximum(i - GRID, 0), 0)),
        out_shape=jax.ShapeDtypeStruct((N, C), jnp.float32),
        scratch_shapes=[
            pltpu.VMEM((N, H), jnp.float32),
            pltpu.VMEM((8, H), jnp.float32),
        ],
    )(partials, partials, xc, w1t, wf1t, be, b1, bf1, gamma2, beta2, wf2, bf2c)


def kernel(x, edge_index, W_edge, b_edge, W_node, b_node, W_cat1, b_cat1,
           W_cat2, b_cat2, W_f1, b_f1, gamma, beta, W_f2, b_f2):
    partials = _seg_sum_sc(W_edge, edge_index)
    xc = _tc_x(x, W_node.T, W_cat2.T, b_node.reshape(1, H), b_cat2.reshape(1, H))
    return _tc_main(
        partials, xc, W_cat1.T, W_f1.T,
        b_edge.reshape(1, H), b_cat1.reshape(1, H), b_f1.reshape(1, H),
        gamma.reshape(1, H), beta.reshape(1, H), W_f2, b_f2.reshape(1, C),
    )


# speedup vs baseline: 1.0128x; 1.0128x over previous
import functools

import jax
import jax.numpy as jnp
from jax import lax
from jax.experimental import pallas as pl
from jax.experimental.pallas import tpu as pltpu
from jax.experimental.pallas import tpu_sc as plsc

N = 10000
E = 320000
D = 128
H = 128
C = 40

NC = 2
NS = 16
NW = NC * NS
CH = 128
NCHUNKS = E // CH
CPT = NCHUNKS // NW
CREM = NCHUNKS - CPT * NW
N_ACC = 10240
ROWS_PER_TILE = N_ACC // NS
ZR = 128
BLK = 1000
GRID = N // BLK


def _seg_sum_sc(W_edge, edge_index):
    mesh = plsc.VectorSubcoreMesh(core_axis_name="c", subcore_axis_name="s")

    @functools.partial(
        pl.kernel,
        out_type=jax.ShapeDtypeStruct((NC, N_ACC, H), jnp.float32),
        mesh=mesh,
        scratch_types=[
            pltpu.VMEM((2, CH), jnp.int32),
            pltpu.VMEM((2, CH), jnp.int32),
            pltpu.VMEM((2, CH), jnp.int32),
            pltpu.VMEM((2, CH), jnp.int32),
            pltpu.VMEM((CH, H), jnp.float32),
            pltpu.VMEM((CH, H), jnp.float32),
            pltpu.VMEM_SHARED((N_ACC, H), jnp.float32),
            pltpu.SemaphoreType.DMA,
            pltpu.SemaphoreType.DMA,
            pltpu.SemaphoreType.DMA,
            pltpu.SemaphoreType.DMA,
            pltpu.SemaphoreType.DMA,
            pltpu.SemaphoreType.DMA,
        ],
    )
    def k(w_hbm, e_hbm, out_hbm, eidx0, eidx1, eidx2, eidx3, rows0, rows1,
          acc, seme0, seme1, seme2, seme3, semg0, semg1):
        c = lax.axis_index("c")
        s = lax.axis_index("s")
        eidx = (eidx0, eidx1, eidx2, eidx3)
        rows = (rows0, rows1)
        seme = (seme0, seme1, seme2, seme3)
        semg = (semg0, semg1)

        wid = c * NS + s
        cbase = wid * CPT + jnp.minimum(wid, CREM)
        nch = CPT + jnp.where(wid < CREM, 1, 0)

        pltpu.sync_copy(e_hbm.at[:, pl.ds(cbase * CH, CH)], eidx0)
        for b in range(1, 4):
            pltpu.async_copy(e_hbm.at[:, pl.ds((cbase + b) * CH, CH)],
                             eidx[b], seme[b])
        pltpu.async_copy(w_hbm.at[eidx0.at[0]], rows0, semg0)

        def zero_row(r, _):
            for j in range(H // 16):
                rows1[r, pl.ds(j * 16, 16)] = jnp.zeros((16,), jnp.float32)
            return 0

        lax.fori_loop(0, CH, zero_row, 0)
        for b in range(ROWS_PER_TILE // ZR):
            pltpu.async_copy(rows1, acc.at[pl.ds(s * ROWS_PER_TILE + b * ZR, ZR)],
                             semg1)
        for b in range(ROWS_PER_TILE // ZR):
            pltpu.make_async_copy(rows1, acc.at[pl.ds(0, ZR)], semg1).wait()
        plsc.subcore_barrier()

        def step(k_i, _):
            b4 = lax.rem(k_i, 4)
            for bb in range(4):
                rb = bb % 2

                @pl.when(b4 == bb)
                def _():
                    @pl.when(k_i + 1 < nch)
                    def _():
                        pltpu.make_async_copy(
                            e_hbm.at[:, pl.ds(0, CH)], eidx[(bb + 1) % 4],
                            seme[(bb + 1) % 4]).wait()
                        pltpu.async_copy(w_hbm.at[eidx[(bb + 1) % 4].at[0]],
                                         rows[1 - rb], semg[1 - rb])

                    pltpu.make_async_copy(w_hbm.at[pl.ds(0, CH)], rows[rb],
                                          semg[rb]).wait()
                    pltpu.sync_copy(rows[rb], acc.at[eidx[bb].at[1]], add=True)

                    @pl.when(k_i + 4 < nch)
                    def _():
                        pltpu.async_copy(
                            e_hbm.at[:, pl.ds((cbase + k_i + 4) * CH, CH)],
                            eidx[bb], seme[bb])
            return 0

        lax.fori_loop(0, nch, step, 0)
        plsc.subcore_barrier()
        pltpu.sync_copy(
            acc.at[pl.ds(s * ROWS_PER_TILE, ROWS_PER_TILE)],
            out_hbm.at[c, pl.ds(s * ROWS_PER_TILE, ROWS_PER_TILE)],
        )

    return k(W_edge, edge_index)


def _tc_x(x, wnt, w2t, bn, b2):
    def body(x_ref, wnt_ref, w2t_ref, bn_ref, b2_ref, o_ref):
        xh = jnp.dot(x_ref[...], wnt_ref[...], preferred_element_type=jnp.float32) + bn_ref[...]
        o_ref[...] = xh + jnp.dot(xh, w2t_ref[...], preferred_element_type=jnp.float32) + b2_ref[...]

    full = lambda i: (0, 0)
    return pl.pallas_call(
        body,
        grid=(GRID,),
        in_specs=[
            pl.BlockSpec((BLK, D), lambda i: (i, 0)),
            pl.BlockSpec((D, H), full),
            pl.BlockSpec((H, H), full),
            pl.BlockSpec((1, H), full),
            pl.BlockSpec((1, H), full),
        ],
        out_specs=pl.BlockSpec((BLK, H), lambda i: (i, 0)),
        out_shape=jax.ShapeDtypeStruct((N, H), jnp.float32),
    )(x, wnt, w2t, bn, b2)


def _tc_main(partials, xc, w1t, wf1t, be, b1, bf1, gamma2, beta2, wf2, bf2c):
    def body(p0_ref, p1_ref, xc_ref, w1t_ref, wf1t_ref, be_ref, b1_ref,
             bf1_ref, g_ref, bb_ref, wf2_ref, bf2_ref, o_ref, h_scr, st_scr):
        i = pl.program_id(0)

        @pl.when(i < GRID)
        def _():
            a = p0_ref[0] + p1_ref[0] + be_ref[...]
            a2 = a + jnp.dot(a, w1t_ref[...], preferred_element_type=jnp.float32) + b1_ref[...]
            out = jnp.maximum(a2 + xc_ref[...], 0.0)
            h1 = jnp.dot(out, wf1t_ref[...], preferred_element_type=jnp.float32) + bf1_ref[...]
            h1 = jnp.maximum(h1, 0.0)
            h_scr[pl.ds(i * BLK, BLK), :] = h1
            upd = jnp.concatenate(
                [jnp.sum(h1, axis=0, keepdims=True),
                 jnp.sum(h1 * h1, axis=0, keepdims=True),
                 jnp.zeros((6, H), jnp.float32)], axis=0)

            @pl.when(i == 0)
            def _():
                st_scr[...] = upd

            @pl.when(i > 0)
            def _():
                st_scr[...] = st_scr[...] + upd

        @pl.when(i >= GRID)
        def _():
            j = i - GRID
            st = st_scr[...]
            mean = st[0:1, :] * (1.0 / N)
            var = st[1:2, :] * (1.0 / N) - mean * mean
            scale = lax.rsqrt(var + 1e-5) * g_ref[...]
            h1 = h_scr[pl.ds(j * BLK, BLK), :]
            hn = (h1 - mean) * scale + bb_ref[...]
            lt = lax.dot_general(hn, wf2_ref[...], (((1,), (1,)), ((), ())),
                                 preferred_element_type=jnp.float32)
            lt = lt + bf2_ref[...]
            m = jnp.max(lt, axis=1, keepdims=True)
            ex = jnp.exp(lt - m)
            lse = jnp.log(jnp.sum(ex, axis=1, keepdims=True))
            o_ref[...] = lt - m - lse

    full = lambda i: (0, 0)
    rowblk = lambda i: (jnp.minimum(i, GRID - 1), 0)
    return pl.pallas_call(
        body,
        grid=(2 * GRID,),
        in_specs=[
            pl.BlockSpec((1, BLK, H), lambda i: (0, jnp.minimum(i, GRID - 1), 0)),
            pl.BlockSpec((1, BLK, H), lambda i: (1, jnp.minimum(i, GRID - 1), 0)),
            pl.BlockSpec((BLK, H), rowblk),
            pl.BlockSpec((H, H), full),
            pl.BlockSpec((H, H), full),
            pl.BlockSpec((1, H), full),
            pl.BlockSpec((1, H), full),
            pl.BlockSpec((1, H), full),
            pl.BlockSpec((1, H), full),
            pl.BlockSpec((1, H), full),
            pl.BlockSpec((C, H), full),
            pl.BlockSpec((1, C), full),
        ],
        out_specs=pl.BlockSpec((BLK, C), lambda i: (jnp.maximum(i - GRID, 0), 0)),
        out_shape=jax.ShapeDtypeStruct((N, C), jnp.float32),
        scratch_shapes=[
            pltpu.VMEM((N, H), jnp.float32),
            pltpu.VMEM((8, H), jnp.float32),
        ],
    )(partials, partials, xc, w1t, wf1t, be, b1, bf1, gamma2, beta2, wf2, bf2c)


def kernel(x, edge_index, W_edge, b_edge, W_node, b_node, W_cat1, b_cat1,
           W_cat2, b_cat2, W_f1, b_f1, gamma, beta, W_f2, b_f2):
    partials = _seg_sum_sc(W_edge, edge_index)
    xc = _tc_x(x, W_node.T, W_cat2.T, b_node.reshape(1, H), b_cat2.reshape(1, H))
    return _tc_main(
        partials, xc, W_cat1.T, W_f1.T,
        b_edge.reshape(1, H), b_cat1.reshape(1, H), b_f1.reshape(1, H),
        gamma.reshape(1, H), beta.reshape(1, H), W_f2, b_f2.reshape(1, C),
    )


# speedup vs baseline: 1.0481x; 1.0349x over previous
import functools

import jax
import jax.numpy as jnp
from jax import lax
from jax.experimental import pallas as pl
from jax.experimental.pallas import tpu as pltpu
from jax.experimental.pallas import tpu_sc as plsc

N = 10000
E = 320000
D = 128
H = 128
C = 40

NC = 2
NS = 16
NW = NC * NS
CH = 128
NCHUNKS = E // CH
CPT = NCHUNKS // NW
CREM = NCHUNKS - CPT * NW
N_ACC = 10240
ROWS_PER_TILE = N_ACC // NS
ZR = 128
BLK = 1000
BLK2 = 1024
GRID = N // BLK


def _seg_sum_sc(W_edge, edge_index):
    mesh = plsc.VectorSubcoreMesh(core_axis_name="c", subcore_axis_name="s")

    @functools.partial(
        pl.kernel,
        out_type=jax.ShapeDtypeStruct((NC, N_ACC, H), jnp.float32),
        mesh=mesh,
        scratch_types=[
            pltpu.VMEM((2, CH), jnp.int32),
            pltpu.VMEM((2, CH), jnp.int32),
            pltpu.VMEM((2, CH), jnp.int32),
            pltpu.VMEM((2, CH), jnp.int32),
            pltpu.VMEM((CH, H), jnp.float32),
            pltpu.VMEM((CH, H), jnp.float32),
            pltpu.VMEM_SHARED((N_ACC, H), jnp.float32),
            pltpu.SemaphoreType.DMA,
            pltpu.SemaphoreType.DMA,
            pltpu.SemaphoreType.DMA,
            pltpu.SemaphoreType.DMA,
            pltpu.SemaphoreType.DMA,
            pltpu.SemaphoreType.DMA,
        ],
    )
    def k(w_hbm, e_hbm, out_hbm, eidx0, eidx1, eidx2, eidx3, rows0, rows1,
          acc, seme0, seme1, seme2, seme3, semg0, semg1):
        c = lax.axis_index("c")
        s = lax.axis_index("s")
        eidx = (eidx0, eidx1, eidx2, eidx3)
        rows = (rows0, rows1)
        seme = (seme0, seme1, seme2, seme3)
        semg = (semg0, semg1)

        wid = c * NS + s
        cbase = wid * CPT + jnp.minimum(wid, CREM)
        nch = CPT + jnp.where(wid < CREM, 1, 0)

        pltpu.sync_copy(e_hbm.at[:, pl.ds(cbase * CH, CH)], eidx0)
        for b in range(1, 4):
            pltpu.async_copy(e_hbm.at[:, pl.ds((cbase + b) * CH, CH)],
                             eidx[b], seme[b])
        pltpu.async_copy(w_hbm.at[eidx0.at[0]], rows0, semg0)

        def zero_row(r, _):
            for j in range(H // 16):
                rows1[r, pl.ds(j * 16, 16)] = jnp.zeros((16,), jnp.float32)
            return 0

        lax.fori_loop(0, CH, zero_row, 0)
        for b in range(ROWS_PER_TILE // ZR):
            pltpu.async_copy(rows1, acc.at[pl.ds(s * ROWS_PER_TILE + b * ZR, ZR)],
                             semg1)
        for b in range(ROWS_PER_TILE // ZR):
            pltpu.make_async_copy(rows1, acc.at[pl.ds(0, ZR)], semg1).wait()
        plsc.subcore_barrier()

        def step(k_i, _):
            b4 = lax.rem(k_i, 4)
            for bb in range(4):
                rb = bb % 2

                @pl.when(b4 == bb)
                def _():
                    @pl.when(k_i + 1 < nch)
                    def _():
                        pltpu.make_async_copy(
                            e_hbm.at[:, pl.ds(0, CH)], eidx[(bb + 1) % 4],
                            seme[(bb + 1) % 4]).wait()
                        pltpu.async_copy(w_hbm.at[eidx[(bb + 1) % 4].at[0]],
                                         rows[1 - rb], semg[1 - rb])

                    pltpu.make_async_copy(w_hbm.at[pl.ds(0, CH)], rows[rb],
                                          semg[rb]).wait()
                    pltpu.sync_copy(rows[rb], acc.at[eidx[bb].at[1]], add=True)

                    @pl.when(k_i + 4 < nch)
                    def _():
                        pltpu.async_copy(
                            e_hbm.at[:, pl.ds((cbase + k_i + 4) * CH, CH)],
                            eidx[bb], seme[bb])
            return 0

        lax.fori_loop(0, nch, step, 0)
        plsc.subcore_barrier()
        pltpu.sync_copy(
            acc.at[pl.ds(s * ROWS_PER_TILE, ROWS_PER_TILE)],
            out_hbm.at[c, pl.ds(s * ROWS_PER_TILE, ROWS_PER_TILE)],
        )

    return k(W_edge, edge_index)


def _tc_x(x, wnt, w2t, bn, b2):
    def body(x_ref, wnt_ref, w2t_ref, bn_ref, b2_ref, o_ref):
        xh = jnp.dot(x_ref[...], wnt_ref[...], preferred_element_type=jnp.float32) + bn_ref[...]
        o_ref[...] = xh + jnp.dot(xh, w2t_ref[...], preferred_element_type=jnp.float32) + b2_ref[...]

    full = lambda i: (0, 0)
    return pl.pallas_call(
        body,
        grid=(GRID,),
        in_specs=[
            pl.BlockSpec((BLK, D), lambda i: (i, 0)),
            pl.BlockSpec((D, H), full),
            pl.BlockSpec((H, H), full),
            pl.BlockSpec((1, H), full),
            pl.BlockSpec((1, H), full),
        ],
        out_specs=pl.BlockSpec((BLK, H), lambda i: (i, 0)),
        out_shape=jax.ShapeDtypeStruct((N, H), jnp.float32),
    )(x, wnt, w2t, bn, b2)


def _tc_main(partials, xc, w1t, wf1t, be, b1, bf1, gamma2, beta2, wf2, bf2c):
    def body(p0_ref, p1_ref, xc_ref, w1t_ref, wf1t_ref, be_ref, b1_ref,
             bf1_ref, g_ref, bb_ref, wf2_ref, bf2_ref, o_ref, h_scr, st_scr):
        i = pl.program_id(0)

        @pl.when(i < GRID)
        def _():
            a = p0_ref[0] + p1_ref[0] + be_ref[...]
            a2 = a + jnp.dot(a, w1t_ref[...], preferred_element_type=jnp.float32) + b1_ref[...]
            out = jnp.maximum(a2 + xc_ref[...], 0.0)
            h1 = jnp.dot(out, wf1t_ref[...], preferred_element_type=jnp.float32) + bf1_ref[...]
            h1 = jnp.maximum(h1, 0.0)
            h_scr[pl.ds(i * BLK, BLK), :] = h1
            upd = jnp.concatenate(
                [jnp.sum(h1, axis=0, keepdims=True),
                 jnp.sum(h1 * h1, axis=0, keepdims=True),
                 jnp.zeros((6, H), jnp.float32)], axis=0)

            @pl.when(i == 0)
            def _():
                st_scr[...] = upd

            @pl.when(i > 0)
            def _():
                st_scr[...] = st_scr[...] + upd

        @pl.when(i == 0)
        def _():
            h_scr[pl.ds(N, N_ACC - N), :] = jnp.zeros((N_ACC - N, H), jnp.float32)

        @pl.when(i >= GRID)
        def _():
            j = i - GRID
            st = st_scr[...]
            mean = st[0:1, :] * (1.0 / N)
            var = st[1:2, :] * (1.0 / N) - mean * mean
            scale = lax.rsqrt(var + 1e-5) * g_ref[...]
            h1 = h_scr[pl.ds(j * BLK2, BLK2), :]
            hn = (h1 - mean) * scale + bb_ref[...]
            lt = lax.dot_general(wf2_ref[...], hn, (((1,), (1,)), ((), ())),
                                 preferred_element_type=jnp.float32)
            lt = lt + bf2_ref[...]
            m = jnp.max(lt, axis=0, keepdims=True)
            ex = jnp.exp(lt - m)
            lse = jnp.log(jnp.sum(ex, axis=0, keepdims=True))
            o_ref[...] = lt - m - lse

    full = lambda i: (0, 0)
    rowblk = lambda i: (jnp.minimum(i, GRID - 1), 0)
    return pl.pallas_call(
        body,
        grid=(2 * GRID,),
        in_specs=[
            pl.BlockSpec((1, BLK, H), lambda i: (0, jnp.minimum(i, GRID - 1), 0)),
            pl.BlockSpec((1, BLK, H), lambda i: (1, jnp.minimum(i, GRID - 1), 0)),
            pl.BlockSpec((BLK, H), rowblk),
            pl.BlockSpec((H, H), full),
            pl.BlockSpec((H, H), full),
            pl.BlockSpec((1, H), full),
            pl.BlockSpec((1, H), full),
            pl.BlockSpec((1, H), full),
            pl.BlockSpec((1, H), full),
            pl.BlockSpec((1, H), full),
            pl.BlockSpec((C, H), full),
            pl.BlockSpec((C, 1), full),
        ],
        out_specs=pl.BlockSpec((C, BLK2), lambda i: (0, jnp.maximum(i - GRID, 0))),
        out_shape=jax.ShapeDtypeStruct((C, N_ACC), jnp.float32),
        scratch_shapes=[
            pltpu.VMEM((N_ACC, H), jnp.float32),
            pltpu.VMEM((8, H), jnp.float32),
        ],
    )(partials, partials, xc, w1t, wf1t, be, b1, bf1, gamma2, beta2, wf2, bf2c)


def kernel(x, edge_index, W_edge, b_edge, W_node, b_node, W_cat1, b_cat1,
           W_cat2, b_cat2, W_f1, b_f1, gamma, beta, W_f2, b_f2):
    partials = _seg_sum_sc(W_edge, edge_index)
    xc = _tc_x(x, W_node.T, W_cat2.T, b_node.reshape(1, H), b_cat2.reshape(1, H))
    outT = _tc_main(
        partials, xc, W_cat1.T, W_f1.T,
        b_edge.reshape(1, H), b_cat1.reshape(1, H), b_f1.reshape(1, H),
        gamma.reshape(1, H), beta.reshape(1, H), W_f2, b_f2.reshape(C, 1),
    )
    return outT[:, :N].T
